# baseline (speedup 1.0000x reference)
import functools

import jax
import jax.numpy as jnp
from jax import lax
from jax.experimental import pallas as pl
from jax.experimental.pallas import tpu as pltpu
from jax.experimental.pallas import tpu_sc as plsc

_N_TOKENS = 8192
_DIM = 2048
_NUM_EXPERT = 16
_TOP_K = 2
_BT = 1024

_NOISE_CACHE = None


def _noise():
    global _NOISE_CACHE
    if _NOISE_CACHE is None:
        _NOISE_CACHE = jax.random.normal(
            jax.random.key(42), (_N_TOKENS, _NUM_EXPERT), dtype=jnp.float32)
    return _NOISE_CACHE


def _logits_body(x_ref, wg_ref, wn_ref, bg_ref, bn_ref, noise_ref,
                 logits_ref):
    xb = x_ref[...]
    gate = lax.dot_general(
        xb, wg_ref[...], (((1,), (1,)), ((), ())),
        preferred_element_type=jnp.float32) + bg_ref[...]
    noisy = lax.dot_general(
        xb, wn_ref[...], (((1,), (1,)), ((), ())),
        preferred_element_type=jnp.float32) + bn_ref[...]
    logits_ref[...] = gate + noise_ref[...] + noisy


def _tc_logits(x, Wg, bg, Wn, bn, noise, n_tok, chunk=0):
    grid = (n_tok // _BT,)
    off = chunk * (n_tok // _BT)
    return pl.pallas_call(
        _logits_body,
        grid=grid,
        in_specs=[
            pl.BlockSpec((_BT, _DIM), lambda i: (i + off, 0)),
            pl.BlockSpec((_NUM_EXPERT, _DIM), lambda i: (0, 0)),
            pl.BlockSpec((_NUM_EXPERT, _DIM), lambda i: (0, 0)),
            pl.BlockSpec((1, _NUM_EXPERT), lambda i: (0, 0)),
            pl.BlockSpec((1, _NUM_EXPERT), lambda i: (0, 0)),
            pl.BlockSpec((_BT, _NUM_EXPERT), lambda i: (i + off, 0)),
        ],
        out_specs=pl.BlockSpec((_BT, _NUM_EXPERT), lambda i: (i, 0)),
        out_shape=jax.ShapeDtypeStruct((n_tok, _NUM_EXPERT),
                                       jnp.float32),
        compiler_params=pltpu.CompilerParams(
            dimension_semantics=("arbitrary",),
        ),
    )(x, Wg, Wn, bg.reshape(1, _NUM_EXPERT), bn.reshape(1, _NUM_EXPERT),
      noise)


_SC_INFO = None


def _sc_router_call(n_tok):
    mesh = plsc.VectorSubcoreMesh(core_axis_name="c", subcore_axis_name="s")
    info = plsc.get_sparse_core_info()
    nw = info.num_cores * info.num_subcores
    t_per_w = n_tok // nw
    n_groups = t_per_w // _NUM_EXPERT

    @functools.partial(
        pl.kernel,
        mesh=mesh,
        out_type=[
            jax.ShapeDtypeStruct((n_tok * _NUM_EXPERT,), jnp.float32),
            jax.ShapeDtypeStruct((n_tok * _TOP_K,), jnp.int32),
        ],
        scratch_types=[
            pltpu.VMEM((t_per_w * _NUM_EXPERT,), jnp.float32),
            pltpu.VMEM((t_per_w * _NUM_EXPERT,), jnp.float32),
            pltpu.VMEM((t_per_w * _TOP_K,), jnp.int32),
        ],
        compiler_params=pltpu.CompilerParams(needs_layout_passes=False),
    )
    def sc_router(logits_hbm, scores_hbm, ids_hbm, lv, sv, idv):
        wid = lax.axis_index("s") * info.num_cores + lax.axis_index("c")
        base = wid * (t_per_w * _NUM_EXPERT)
        pltpu.sync_copy(logits_hbm.at[pl.ds(base, t_per_w * _NUM_EXPERT)],
                        lv)

        lanes = lax.iota(jnp.int32, _NUM_EXPERT)
        neg_inf = jnp.full((_NUM_EXPERT,), -jnp.inf, jnp.float32)
        zeros_f = jnp.zeros((_NUM_EXPERT,), jnp.float32)
        zeros_i = jnp.zeros((_NUM_EXPERT,), jnp.int32)

        def group(g, carry):
            row0 = g * (_NUM_EXPERT * _NUM_EXPERT)
            flat0 = row0 + lanes * _NUM_EXPERT
            for j in range(_NUM_EXPERT):
                sv[pl.ds(row0 + j * _NUM_EXPERT, _NUM_EXPERT)] = zeros_f
            m1, m2 = neg_inf, neg_inf
            i1, i2 = zeros_i, zeros_i
            for e in range(_NUM_EXPERT):
                e_vec = jnp.full((_NUM_EXPERT,), e, jnp.int32)
                v = plsc.load_gather(lv, [flat0 + e])
                gt1 = v > m1
                gt2 = v > m2
                i2 = jnp.where(gt1, i1, jnp.where(gt2, e_vec, i2))
                m2 = jnp.where(gt1, m1, jnp.where(gt2, v, m2))
                i1 = jnp.where(gt1, e_vec, i1)
                m1 = jnp.where(gt1, v, m1)
            ed = jnp.exp(m2 - m1)
            p1 = 1.0 / (1.0 + ed)
            p2 = ed * p1
            plsc.store_scatter(sv, [flat0 + i1], p1)
            plsc.store_scatter(sv, [flat0 + i2], p2)
            pair0 = g * (_NUM_EXPERT * _TOP_K) + lanes * _TOP_K
            plsc.store_scatter(idv, [pair0], i1)
            plsc.store_scatter(idv, [pair0 + 1], i2)
            return carry

        lax.fori_loop(0, n_groups, group, 0)
        pltpu.sync_copy(sv, scores_hbm.at[pl.ds(base, t_per_w * _NUM_EXPERT)])
        pltpu.sync_copy(
            idv, ids_hbm.at[pl.ds(wid * t_per_w * _TOP_K, t_per_w * _TOP_K)])

    return sc_router


def kernel(x, Wg, bg, Wn, bn):
    noise = _noise()
    half = _N_TOKENS // 2
    sc = _sc_router_call(half)
    la = _tc_logits(x, Wg, bg, Wn, bn, noise, half, 0)
    sa, ia = sc(la.reshape(-1))
    lb = _tc_logits(x, Wg, bg, Wn, bn, noise, half, 1)
    sb, ib = sc(lb.reshape(-1))
    scores = jnp.concatenate(
        [sa.reshape(half, _NUM_EXPERT), sb.reshape(half, _NUM_EXPERT)])
    ids = jnp.concatenate(
        [ia.reshape(half, _TOP_K), ib.reshape(half, _TOP_K)])
    return scores, ids


# speedup vs baseline: 1.0068x; 1.0068x over previous
import functools

import jax
import jax.numpy as jnp
from jax import lax
from jax.experimental import pallas as pl
from jax.experimental.pallas import tpu as pltpu
from jax.experimental.pallas import tpu_sc as plsc

_N_TOKENS = 8192
_DIM = 2048
_NUM_EXPERT = 16
_TOP_K = 2
_BT = 1024

_NOISE_CACHE = None


def _noise():
    global _NOISE_CACHE
    if _NOISE_CACHE is None:
        _NOISE_CACHE = jax.random.normal(
            jax.random.key(42), (_N_TOKENS, _NUM_EXPERT), dtype=jnp.float32)
    return _NOISE_CACHE


def _logits_body(x_ref, wg_ref, wn_ref, bg_ref, bn_ref, noise_ref,
                 logits_ref):
    xb = x_ref[...]
    gate = lax.dot_general(
        xb, wg_ref[...], (((1,), (1,)), ((), ())),
        preferred_element_type=jnp.float32) + bg_ref[...]
    noisy = lax.dot_general(
        xb, wn_ref[...], (((1,), (1,)), ((), ())),
        preferred_element_type=jnp.float32) + bn_ref[...]
    logits_ref[...] = gate + noise_ref[...] + noisy


def _tc_logits(x, Wg, bg, Wn, bn, noise, n_tok, chunk=0):
    grid = (n_tok // _BT,)
    off = chunk * (n_tok // _BT)
    return pl.pallas_call(
        _logits_body,
        grid=grid,
        in_specs=[
            pl.BlockSpec((_BT, _DIM), lambda i: (i + off, 0)),
            pl.BlockSpec((_NUM_EXPERT, _DIM), lambda i: (0, 0)),
            pl.BlockSpec((_NUM_EXPERT, _DIM), lambda i: (0, 0)),
            pl.BlockSpec((1, _NUM_EXPERT), lambda i: (0, 0)),
            pl.BlockSpec((1, _NUM_EXPERT), lambda i: (0, 0)),
            pl.BlockSpec((_BT, _NUM_EXPERT), lambda i: (i + off, 0)),
        ],
        out_specs=pl.BlockSpec((_BT, _NUM_EXPERT), lambda i: (i, 0)),
        out_shape=jax.ShapeDtypeStruct((n_tok, _NUM_EXPERT),
                                       jnp.float32),
        compiler_params=pltpu.CompilerParams(
            dimension_semantics=("arbitrary",),
        ),
    )(x, Wg, Wn, bg.reshape(1, _NUM_EXPERT), bn.reshape(1, _NUM_EXPERT),
      noise)


def _fused_body(x_ref, wg_ref, wn_ref, bg_ref, bn_ref, noise_ref,
                scores_ref, ids_ref):
    xb = x_ref[...]
    gate = lax.dot_general(
        xb, wg_ref[...], (((1,), (1,)), ((), ())),
        preferred_element_type=jnp.float32) + bg_ref[...]
    noisy = lax.dot_general(
        xb, wn_ref[...], (((1,), (1,)), ((), ())),
        preferred_element_type=jnp.float32) + bn_ref[...]
    logits = gate + noise_ref[...] + noisy

    iota = lax.broadcasted_iota(jnp.int32, logits.shape, 1)
    neg_inf = jnp.float32(-jnp.inf)
    m1 = jnp.max(logits, axis=1, keepdims=True)
    i1 = jnp.min(jnp.where(logits == m1, iota, _NUM_EXPERT),
                 axis=1, keepdims=True)
    rest = jnp.where(iota == i1, neg_inf, logits)
    m2 = jnp.max(rest, axis=1, keepdims=True)
    i2 = jnp.min(jnp.where(rest == m2, iota, _NUM_EXPERT),
                 axis=1, keepdims=True)
    ed = jnp.exp(m2 - m1)
    p1 = 1.0 / (1.0 + ed)
    p2 = ed * p1
    scores_ref[...] = (jnp.where(iota == i1, p1, 0.0)
                       + jnp.where(iota == i2, p2, 0.0))
    ids_ref[...] = jnp.concatenate([i1, i2], axis=1)


def _tc_fused(x, Wg, bg, Wn, bn, noise, n_tok, chunk=0):
    grid = (n_tok // _BT,)
    off = chunk * (n_tok // _BT)
    return pl.pallas_call(
        _fused_body,
        grid=grid,
        in_specs=[
            pl.BlockSpec((_BT, _DIM), lambda i: (i + off, 0)),
            pl.BlockSpec((_NUM_EXPERT, _DIM), lambda i: (0, 0)),
            pl.BlockSpec((_NUM_EXPERT, _DIM), lambda i: (0, 0)),
            pl.BlockSpec((1, _NUM_EXPERT), lambda i: (0, 0)),
            pl.BlockSpec((1, _NUM_EXPERT), lambda i: (0, 0)),
            pl.BlockSpec((_BT, _NUM_EXPERT), lambda i: (i + off, 0)),
        ],
        out_specs=[
            pl.BlockSpec((_BT, _NUM_EXPERT), lambda i: (i, 0)),
            pl.BlockSpec((_BT, _TOP_K), lambda i: (i, 0)),
        ],
        out_shape=[
            jax.ShapeDtypeStruct((n_tok, _NUM_EXPERT), jnp.float32),
            jax.ShapeDtypeStruct((n_tok, _TOP_K), jnp.int32),
        ],
        compiler_params=pltpu.CompilerParams(
            dimension_semantics=("arbitrary",),
        ),
    )(x, Wg, Wn, bg.reshape(1, _NUM_EXPERT), bn.reshape(1, _NUM_EXPERT),
      noise)


def _sc_router_call(n_tok):
    mesh = plsc.VectorSubcoreMesh(core_axis_name="c", subcore_axis_name="s")
    info = plsc.get_sparse_core_info()
    nw = info.num_cores * info.num_subcores
    t_per_w = n_tok // nw
    n_groups = t_per_w // _NUM_EXPERT

    @functools.partial(
        pl.kernel,
        mesh=mesh,
        out_type=[
            jax.ShapeDtypeStruct((n_tok * _NUM_EXPERT,), jnp.float32),
            jax.ShapeDtypeStruct((n_tok * _TOP_K,), jnp.int32),
        ],
        scratch_types=[
            pltpu.VMEM((t_per_w * _NUM_EXPERT,), jnp.float32),
            pltpu.VMEM((t_per_w * _NUM_EXPERT,), jnp.float32),
            pltpu.VMEM((t_per_w * _TOP_K,), jnp.int32),
        ],
        compiler_params=pltpu.CompilerParams(needs_layout_passes=False),
    )
    def sc_router(logits_hbm, scores_hbm, ids_hbm, lv, sv, idv):
        wid = lax.axis_index("s") * info.num_cores + lax.axis_index("c")
        base = wid * (t_per_w * _NUM_EXPERT)
        pltpu.sync_copy(logits_hbm.at[pl.ds(base, t_per_w * _NUM_EXPERT)],
                        lv)

        lanes = lax.iota(jnp.int32, _NUM_EXPERT)
        neg_inf = jnp.full((_NUM_EXPERT,), -jnp.inf, jnp.float32)
        zeros_f = jnp.zeros((_NUM_EXPERT,), jnp.float32)
        zeros_i = jnp.zeros((_NUM_EXPERT,), jnp.int32)

        def group(g, carry):
            row0 = g * (_NUM_EXPERT * _NUM_EXPERT)
            flat0 = row0 + lanes * _NUM_EXPERT
            for j in range(_NUM_EXPERT):
                sv[pl.ds(row0 + j * _NUM_EXPERT, _NUM_EXPERT)] = zeros_f
            m1, m2 = neg_inf, neg_inf
            i1, i2 = zeros_i, zeros_i
            for e in range(_NUM_EXPERT):
                e_vec = jnp.full((_NUM_EXPERT,), e, jnp.int32)
                v = plsc.load_gather(lv, [flat0 + e])
                gt1 = v > m1
                gt2 = v > m2
                i2 = jnp.where(gt1, i1, jnp.where(gt2, e_vec, i2))
                m2 = jnp.where(gt1, m1, jnp.where(gt2, v, m2))
                i1 = jnp.where(gt1, e_vec, i1)
                m1 = jnp.where(gt1, v, m1)
            ed = jnp.exp(m2 - m1)
            p1 = 1.0 / (1.0 + ed)
            p2 = ed * p1
            plsc.store_scatter(sv, [flat0 + i1], p1)
            plsc.store_scatter(sv, [flat0 + i2], p2)
            pair0 = g * (_NUM_EXPERT * _TOP_K) + lanes * _TOP_K
            plsc.store_scatter(idv, [pair0], i1)
            plsc.store_scatter(idv, [pair0 + 1], i2)
            return carry

        lax.fori_loop(0, n_groups, group, 0)
        pltpu.sync_copy(sv, scores_hbm.at[pl.ds(base, t_per_w * _NUM_EXPERT)])
        pltpu.sync_copy(
            idv, ids_hbm.at[pl.ds(wid * t_per_w * _TOP_K, t_per_w * _TOP_K)])

    return sc_router


def kernel(x, Wg, bg, Wn, bn):
    noise = _noise()
    half = _N_TOKENS // 2
    la = _tc_logits(x, Wg, bg, Wn, bn, noise, half, 0)
    sa, ia = _sc_router_call(half)(la.reshape(-1))
    sb, ib = _tc_fused(x, Wg, bg, Wn, bn, noise, half, 1)
    scores = jnp.concatenate(
        [sa.reshape(half, _NUM_EXPERT), sb])
    ids = jnp.concatenate(
        [ia.reshape(half, _TOP_K), ib])
    return scores, ids


# speedup vs baseline: 1.3816x; 1.3724x over previous
import functools

import jax
import jax.numpy as jnp
from jax import lax
from jax.experimental import pallas as pl
from jax.experimental.pallas import tpu as pltpu

_N_TOKENS = 8192
_DIM = 2048
_NUM_EXPERT = 16
_TOP_K = 2
_BT = 2048

_NOISE_CACHE = None


def _noise():
    global _NOISE_CACHE
    if _NOISE_CACHE is None:
        _NOISE_CACHE = jax.random.normal(
            jax.random.key(42), (_N_TOKENS, _NUM_EXPERT), dtype=jnp.float32)
    return _NOISE_CACHE


def _router_body(x_ref, wg_ref, wn_ref, bg_ref, bn_ref, noise_ref,
                 scores_ref, ids_ref):
    xb = x_ref[...]
    gate = lax.dot_general(
        xb, wg_ref[...], (((1,), (1,)), ((), ())),
        preferred_element_type=jnp.float32) + bg_ref[...]
    noisy = lax.dot_general(
        xb, wn_ref[...], (((1,), (1,)), ((), ())),
        preferred_element_type=jnp.float32) + bn_ref[...]
    logits = gate + noise_ref[...] + noisy

    iota = lax.broadcasted_iota(jnp.int32, logits.shape, 1)
    neg_inf = jnp.float32(-jnp.inf)
    m1 = jnp.max(logits, axis=1, keepdims=True)
    i1 = jnp.min(jnp.where(logits == m1, iota, _NUM_EXPERT),
                 axis=1, keepdims=True)
    rest = jnp.where(iota == i1, neg_inf, logits)
    m2 = jnp.max(rest, axis=1, keepdims=True)
    i2 = jnp.min(jnp.where(rest == m2, iota, _NUM_EXPERT),
                 axis=1, keepdims=True)

    ed = jnp.exp(m2 - m1)
    denom = 1.0 / (1.0 + ed)
    p1 = denom
    p2 = ed * denom
    scores_ref[...] = (jnp.where(iota == i1, p1, 0.0)
                       + jnp.where(iota == i2, p2, 0.0))
    ids_ref[...] = jnp.concatenate([i1, i2], axis=1)


@functools.partial(jax.jit, static_argnums=())
def _router(x, Wg, bg, Wn, bn, noise):
    grid = (_N_TOKENS // _BT,)
    scores, ids = pl.pallas_call(
        _router_body,
        grid=grid,
        in_specs=[
            pl.BlockSpec((_BT, _DIM), lambda i: (i, 0)),
            pl.BlockSpec((_NUM_EXPERT, _DIM), lambda i: (0, 0)),
            pl.BlockSpec((_NUM_EXPERT, _DIM), lambda i: (0, 0)),
            pl.BlockSpec((1, _NUM_EXPERT), lambda i: (0, 0)),
            pl.BlockSpec((1, _NUM_EXPERT), lambda i: (0, 0)),
            pl.BlockSpec((_BT, _NUM_EXPERT), lambda i: (i, 0)),
        ],
        out_specs=[
            pl.BlockSpec((_BT, _NUM_EXPERT), lambda i: (i, 0)),
            pl.BlockSpec((_BT, _TOP_K), lambda i: (i, 0)),
        ],
        out_shape=[
            jax.ShapeDtypeStruct((_N_TOKENS, _NUM_EXPERT), jnp.float32),
            jax.ShapeDtypeStruct((_N_TOKENS, _TOP_K), jnp.int32),
        ],
        compiler_params=pltpu.CompilerParams(
            dimension_semantics=("arbitrary",),
        ),
    )(x, Wg, Wn, bg.reshape(1, _NUM_EXPERT), bn.reshape(1, _NUM_EXPERT),
      noise)
    return scores, ids


def kernel(x, Wg, bg, Wn, bn):
    return _router(x, Wg, bg, Wn, bn, _noise())
